# baseline (speedup 1.0000x reference)
import functools

import jax
import jax.numpy as jnp
from jax import lax
from jax.experimental import pallas as pl
from jax.experimental.pallas import tpu as pltpu
from jax.experimental.pallas import tpu_sc as plsc

BATCH = 4096
N_FIELDS = 26
HIST = 20
VOCAB = 100000
DIM = 32

NC = 2
NS = 16
NW = NC * NS

NUM_BAGS = BATCH * N_FIELDS
NUM_ROWS = NUM_BAGS * HIST
BAGS_PER_W = NUM_BAGS // NW
ROWS_PER_W = BAGS_PER_W * HIST

CHUNK_BAGS = 64
CHUNK_ROWS = CHUNK_BAGS * HIST
CHUNKS_PER_W = BAGS_PER_W // CHUNK_BAGS
IDX_W = 128
N_GATHERS = CHUNK_ROWS // IDX_W


TVB = 768
TCHUNKS = VOCAB // TVB
TTAIL0 = TCHUNKS * TVB
TTAIL1 = TTAIL0 + 128
NU_A = N_FIELDS * TCHUNKS
NK_PAIRS = ((NU_A + NW - 1) // NW + 1) // 2


def _transpose_body(
    tbl_hbm, out_hbm, slab0, slab1, outb0, outb1, slab_b, outb_b, slab_c, outb_c,
    si0, si1, so0, so1,
):
    wid = lax.axis_index("s") * NC + lax.axis_index("c")
    iota = lax.iota(jnp.int32, 16)
    d_lo = iota
    d_hi = iota + 16
    rot = [(iota + r) % 16 for r in range(16)]
    oidx = [rot[r] * DIM + iota for r in range(16)]
    nu_w = (NU_A - 1 - wid) // NW + 1

    def unit_fv(k):
        u = wid + k * NW
        return u // TCHUNKS, (u % TCHUNKS) * TVB

    def fetch(k, slab, sem):
        f, v0 = unit_fv(k)
        pltpu.async_copy(tbl_hbm.at[f, :, pl.ds(v0, TVB)], slab, sem)

    def drain_in(slab, sem):
        pltpu.make_async_copy(tbl_hbm.at[0, :, pl.ds(0, TVB)], slab, sem).wait()

    def drain_out(outb, sem):
        pltpu.make_async_copy(out_hbm.at[pl.ds(0, TVB * DIM)], outb, sem).wait()

    def transpose_cols(vb, slab, outb):
        def j_body(j, carry):
            vbase = j * 16
            obase = vbase * DIM
            for r in range(16):
                vs = rot[r] + vbase
                g0 = plsc.load_gather(slab, [d_lo, vs])
                plsc.store_scatter(outb, [oidx[r] + obase], g0)
                g1 = plsc.load_gather(slab, [d_hi, vs])
                plsc.store_scatter(outb, [oidx[r] + (obase + 16)], g1)
            return carry

        lax.fori_loop(0, vb // 16, j_body, 0, unroll=2)

    def out_copy(k, outb, sem):
        f, v0 = unit_fv(k)
        e0 = pl.multiple_of((f * VOCAB + v0) * DIM, 8)
        pltpu.async_copy(outb, out_hbm.at[pl.ds(e0, TVB * DIM)], sem)

    def stage(k, slab, si, outb, so):
        @pl.when(k < nu_w)
        def _():
            drain_in(slab, si)

            @pl.when(k >= 2)
            def _():
                drain_out(outb, so)

            transpose_cols(TVB, slab, outb)
            out_copy(k, outb, so)

    @pl.when(nu_w > 0)
    def _():
        fetch(0, slab0, si0)

    def pair_body(kk, carry):
        k0 = kk * 2

        @pl.when(k0 + 1 < nu_w)
        def _():
            fetch(k0 + 1, slab1, si1)

        stage(k0, slab0, si0, outb0, so0)

        @pl.when(k0 + 2 < nu_w)
        def _():
            fetch(k0 + 2, slab0, si0)

        stage(k0 + 1, slab1, si1, outb1, so1)
        return carry

    lax.fori_loop(0, NK_PAIRS, pair_body, 0)

    @pl.when(nu_w >= 1)
    def _():
        drain_out(outb0, so0)

    @pl.when(nu_w >= 2)
    def _():
        drain_out(outb1, so1)

    @pl.when(wid < N_FIELDS)
    def _():
        for v0, vb, slab, outb in (
            (TTAIL0, 128, slab_b, outb_b),
            (TTAIL1, 32, slab_c, outb_c),
        ):
            pltpu.sync_copy(tbl_hbm.at[wid, :, pl.ds(v0, vb)], slab)
            transpose_cols(vb, slab, outb)
            e0 = pl.multiple_of((wid * VOCAB + v0) * DIM, 8)
            pltpu.sync_copy(outb, out_hbm.at[pl.ds(e0, vb * DIM)])


_transpose = pl.kernel(
    _transpose_body,
    out_type=jax.ShapeDtypeStruct((N_FIELDS * VOCAB * DIM,), jnp.float32),
    mesh=plsc.VectorSubcoreMesh(
        core_axis_name="c", subcore_axis_name="s", num_cores=NC, num_subcores=NS
    ),
    scratch_types=[
        pltpu.VMEM((DIM, TVB), jnp.float32),
        pltpu.VMEM((DIM, TVB), jnp.float32),
        pltpu.VMEM((TVB * DIM,), jnp.float32),
        pltpu.VMEM((TVB * DIM,), jnp.float32),
        pltpu.VMEM((DIM, 128), jnp.float32),
        pltpu.VMEM((128 * DIM,), jnp.float32),
        pltpu.VMEM((DIM, 32), jnp.float32),
        pltpu.VMEM((32 * DIM,), jnp.float32),
        pltpu.SemaphoreType.DMA,
        pltpu.SemaphoreType.DMA,
        pltpu.SemaphoreType.DMA,
        pltpu.SemaphoreType.DMA,
    ],
    compiler_params=pltpu.CompilerParams(needs_layout_passes=False),
)


def _lookup_body(
    table_hbm, idx_hbm, out_hbm, idx0, idx1, rows0, rows1, out_v, sem0, sem1
):
    wid = lax.axis_index("s") * NC + lax.axis_index("c")
    row_base = wid * ROWS_PER_W
    bag_base = wid * BAGS_PER_W

    def fetch(c, idx_v, sem):
        pltpu.sync_copy(idx_hbm.at[pl.ds(row_base + c * CHUNK_ROWS, CHUNK_ROWS)], idx_v)
        for j in range(N_GATHERS):
            pltpu.async_copy(
                table_hbm.at[idx_v.at[pl.ds(j * IDX_W, IDX_W)]],
                rows0.at[pl.ds(j * IDX_W, IDX_W)] if idx_v is idx0
                else rows1.at[pl.ds(j * IDX_W, IDX_W)],
                sem,
            )

    def drain(rows_v, sem):
        pltpu.make_async_copy(
            table_hbm.at[pl.ds(0, CHUNK_ROWS)], rows_v, sem
        ).wait()

    def pool_store(c, rows_v):
        def bag_body(g, carry2):
            base = g * HIST
            a0 = rows_v[base, pl.ds(0, 16)]
            a1 = rows_v[base, pl.ds(16, 16)]
            for h in range(1, HIST):
                a0 = a0 + rows_v[base + h, pl.ds(0, 16)]
                a1 = a1 + rows_v[base + h, pl.ds(16, 16)]
            out_v[g, pl.ds(0, 16)] = a0
            out_v[g, pl.ds(16, 16)] = a1
            return carry2

        lax.fori_loop(0, CHUNK_BAGS, bag_body, 0)
        pltpu.sync_copy(out_v, out_hbm.at[pl.ds(bag_base + c * CHUNK_BAGS, CHUNK_BAGS)])

    fetch(0, idx0, sem0)

    def pair_body(cc, carry):
        c = cc * 2
        fetch(c + 1, idx1, sem1)
        drain(rows0, sem0)
        pool_store(c, rows0)

        @pl.when(cc + 1 < CHUNKS_PER_W // 2)
        def _():
            fetch(c + 2, idx0, sem0)

        drain(rows1, sem1)
        pool_store(c + 1, rows1)
        return carry

    lax.fori_loop(0, CHUNKS_PER_W // 2, pair_body, 0)


_lookup = pl.kernel(
    _lookup_body,
    out_type=jax.ShapeDtypeStruct((NUM_BAGS, DIM), jnp.float32),
    mesh=plsc.VectorSubcoreMesh(
        core_axis_name="c", subcore_axis_name="s", num_cores=NC, num_subcores=NS
    ),
    scratch_types=[
        pltpu.VMEM((CHUNK_ROWS,), jnp.int32),
        pltpu.VMEM((CHUNK_ROWS,), jnp.int32),
        pltpu.VMEM((CHUNK_ROWS, DIM), jnp.float32),
        pltpu.VMEM((CHUNK_ROWS, DIM), jnp.float32),
        pltpu.VMEM((CHUNK_BAGS, DIM), jnp.float32),
        pltpu.SemaphoreType.DMA,
        pltpu.SemaphoreType.DMA,
    ],
    compiler_params=pltpu.CompilerParams(use_tc_tiling_on_sc=False),
)


@jax.jit
def kernel(indices, tables):
    field_off = (jnp.arange(N_FIELDS, dtype=jnp.int32) * VOCAB)[None, :, None]
    flat_idx = (indices.astype(jnp.int32) + field_off).reshape(NUM_ROWS)
    tables_t = jnp.swapaxes(tables, 1, 2)
    flat_table = _transpose(tables_t).reshape(N_FIELDS * VOCAB, DIM)
    out = _lookup(flat_table, flat_idx)
    return out.reshape(BATCH, N_FIELDS * DIM)


# speedup vs baseline: 1.1009x; 1.1009x over previous
import functools

import jax
import jax.numpy as jnp
from jax import lax
from jax.experimental import pallas as pl
from jax.experimental.pallas import tpu as pltpu
from jax.experimental.pallas import tpu_sc as plsc

BATCH = 4096
N_FIELDS = 26
HIST = 20
VOCAB = 100000
DIM = 32

NC = 2
NS = 16
NW = NC * NS

NUM_BAGS = BATCH * N_FIELDS
NUM_ROWS = NUM_BAGS * HIST
BAGS_PER_W = NUM_BAGS // NW
ROWS_PER_W = BAGS_PER_W * HIST

CHUNK_BAGS = 64
CHUNK_ROWS = CHUNK_BAGS * HIST
CHUNKS_PER_W = BAGS_PER_W // CHUNK_BAGS
IDX_W = 128
N_GATHERS = CHUNK_ROWS // IDX_W


TVB = 768
TCHUNKS = VOCAB // TVB
TTAIL0 = TCHUNKS * TVB
TTAIL1 = TTAIL0 + 128
NU_A = N_FIELDS * TCHUNKS
NK_PAIRS = ((NU_A + NW - 1) // NW + 1) // 2


def _transpose_body(
    tbl_hbm, out_hbm, slab0, slab1, outb0, outb1, slab_b, outb_b, slab_c, outb_c,
    si0, si1, so0, so1,
):
    wid = lax.axis_index("s") * NC + lax.axis_index("c")
    iota = lax.iota(jnp.int32, 16)
    d_lo = iota
    d_hi = iota + 16
    rot = [(iota + r) % 16 for r in range(16)]
    oidx = [rot[r] * DIM + iota for r in range(16)]
    nu_w = (NU_A - 1 - wid) // NW + 1

    def unit_fv(k):
        u = wid + k * NW
        return u // TCHUNKS, (u % TCHUNKS) * TVB

    def fetch(k, slab, sem):
        f, v0 = unit_fv(k)
        pltpu.async_copy(tbl_hbm.at[f, :, pl.ds(v0, TVB)], slab, sem)

    def drain_in(slab, sem):
        pltpu.make_async_copy(tbl_hbm.at[0, :, pl.ds(0, TVB)], slab, sem).wait()

    def drain_out(outb, sem):
        pltpu.make_async_copy(out_hbm.at[pl.ds(0, TVB * DIM)], outb, sem).wait()

    def transpose_cols(vb, slab, outb):
        def j_body(j, carry):
            vbase = j * 16
            obase = vbase * DIM
            for r in range(16):
                vs = rot[r] + vbase
                g0 = plsc.load_gather(slab, [d_lo, vs])
                plsc.store_scatter(outb, [oidx[r] + obase], g0)
                g1 = plsc.load_gather(slab, [d_hi, vs])
                plsc.store_scatter(outb, [oidx[r] + (obase + 16)], g1)
            return carry

        lax.fori_loop(0, vb // 16, j_body, 0, unroll=4)

    def out_copy(k, outb, sem):
        f, v0 = unit_fv(k)
        e0 = pl.multiple_of((f * VOCAB + v0) * DIM, 8)
        pltpu.async_copy(outb, out_hbm.at[pl.ds(e0, TVB * DIM)], sem)

    def stage(k, slab, si, outb, so):
        @pl.when(k < nu_w)
        def _():
            drain_in(slab, si)

            @pl.when(k >= 2)
            def _():
                drain_out(outb, so)

            transpose_cols(TVB, slab, outb)
            out_copy(k, outb, so)

    @pl.when(nu_w > 0)
    def _():
        fetch(0, slab0, si0)

    def pair_body(kk, carry):
        k0 = kk * 2

        @pl.when(k0 + 1 < nu_w)
        def _():
            fetch(k0 + 1, slab1, si1)

        stage(k0, slab0, si0, outb0, so0)

        @pl.when(k0 + 2 < nu_w)
        def _():
            fetch(k0 + 2, slab0, si0)

        stage(k0 + 1, slab1, si1, outb1, so1)
        return carry

    lax.fori_loop(0, NK_PAIRS, pair_body, 0)

    @pl.when(nu_w >= 1)
    def _():
        drain_out(outb0, so0)

    @pl.when(nu_w >= 2)
    def _():
        drain_out(outb1, so1)

    @pl.when(wid < N_FIELDS)
    def _():
        for v0, vb, slab, outb in (
            (TTAIL0, 128, slab_b, outb_b),
            (TTAIL1, 32, slab_c, outb_c),
        ):
            pltpu.sync_copy(tbl_hbm.at[wid, :, pl.ds(v0, vb)], slab)
            transpose_cols(vb, slab, outb)
            e0 = pl.multiple_of((wid * VOCAB + v0) * DIM, 8)
            pltpu.sync_copy(outb, out_hbm.at[pl.ds(e0, vb * DIM)])


_transpose = pl.kernel(
    _transpose_body,
    out_type=jax.ShapeDtypeStruct((N_FIELDS * VOCAB * DIM,), jnp.float32),
    mesh=plsc.VectorSubcoreMesh(
        core_axis_name="c", subcore_axis_name="s", num_cores=NC, num_subcores=NS
    ),
    scratch_types=[
        pltpu.VMEM((DIM, TVB), jnp.float32),
        pltpu.VMEM((DIM, TVB), jnp.float32),
        pltpu.VMEM((TVB * DIM,), jnp.float32),
        pltpu.VMEM((TVB * DIM,), jnp.float32),
        pltpu.VMEM((DIM, 128), jnp.float32),
        pltpu.VMEM((128 * DIM,), jnp.float32),
        pltpu.VMEM((DIM, 32), jnp.float32),
        pltpu.VMEM((32 * DIM,), jnp.float32),
        pltpu.SemaphoreType.DMA,
        pltpu.SemaphoreType.DMA,
        pltpu.SemaphoreType.DMA,
        pltpu.SemaphoreType.DMA,
    ],
    compiler_params=pltpu.CompilerParams(needs_layout_passes=False),
)


def _lookup_body(
    table_hbm, idx_hbm, out_hbm, idx0, idx1, rows0, rows1, out_v, sem0, sem1
):
    wid = lax.axis_index("s") * NC + lax.axis_index("c")
    row_base = wid * ROWS_PER_W
    bag_base = wid * BAGS_PER_W

    def fetch(c, idx_v, sem):
        pltpu.sync_copy(idx_hbm.at[pl.ds(row_base + c * CHUNK_ROWS, CHUNK_ROWS)], idx_v)
        for j in range(N_GATHERS):
            pltpu.async_copy(
                table_hbm.at[idx_v.at[pl.ds(j * IDX_W, IDX_W)]],
                rows0.at[pl.ds(j * IDX_W, IDX_W)] if idx_v is idx0
                else rows1.at[pl.ds(j * IDX_W, IDX_W)],
                sem,
            )

    def drain(rows_v, sem):
        pltpu.make_async_copy(
            table_hbm.at[pl.ds(0, CHUNK_ROWS)], rows_v, sem
        ).wait()

    def pool_store(c, rows_v):
        def bag_body(g, carry2):
            base = g * HIST
            a0 = rows_v[base, pl.ds(0, 16)]
            a1 = rows_v[base, pl.ds(16, 16)]
            for h in range(1, HIST):
                a0 = a0 + rows_v[base + h, pl.ds(0, 16)]
                a1 = a1 + rows_v[base + h, pl.ds(16, 16)]
            out_v[g, pl.ds(0, 16)] = a0
            out_v[g, pl.ds(16, 16)] = a1
            return carry2

        lax.fori_loop(0, CHUNK_BAGS, bag_body, 0)
        pltpu.sync_copy(out_v, out_hbm.at[pl.ds(bag_base + c * CHUNK_BAGS, CHUNK_BAGS)])

    fetch(0, idx0, sem0)

    def pair_body(cc, carry):
        c = cc * 2
        fetch(c + 1, idx1, sem1)
        drain(rows0, sem0)
        pool_store(c, rows0)

        @pl.when(cc + 1 < CHUNKS_PER_W // 2)
        def _():
            fetch(c + 2, idx0, sem0)

        drain(rows1, sem1)
        pool_store(c + 1, rows1)
        return carry

    lax.fori_loop(0, CHUNKS_PER_W // 2, pair_body, 0)


_lookup = pl.kernel(
    _lookup_body,
    out_type=jax.ShapeDtypeStruct((NUM_BAGS, DIM), jnp.float32),
    mesh=plsc.VectorSubcoreMesh(
        core_axis_name="c", subcore_axis_name="s", num_cores=NC, num_subcores=NS
    ),
    scratch_types=[
        pltpu.VMEM((CHUNK_ROWS,), jnp.int32),
        pltpu.VMEM((CHUNK_ROWS,), jnp.int32),
        pltpu.VMEM((CHUNK_ROWS, DIM), jnp.float32),
        pltpu.VMEM((CHUNK_ROWS, DIM), jnp.float32),
        pltpu.VMEM((CHUNK_BAGS, DIM), jnp.float32),
        pltpu.SemaphoreType.DMA,
        pltpu.SemaphoreType.DMA,
    ],
    compiler_params=pltpu.CompilerParams(use_tc_tiling_on_sc=False),
)


@jax.jit
def kernel(indices, tables):
    field_off = (jnp.arange(N_FIELDS, dtype=jnp.int32) * VOCAB)[None, :, None]
    flat_idx = (indices.astype(jnp.int32) + field_off).reshape(NUM_ROWS)
    tables_t = jnp.swapaxes(tables, 1, 2)
    flat_table = _transpose(tables_t).reshape(N_FIELDS * VOCAB, DIM)
    out = _lookup(flat_table, flat_idx)
    return out.reshape(BATCH, N_FIELDS * DIM)


# speedup vs baseline: 1.1012x; 1.0003x over previous
import functools

import jax
import jax.numpy as jnp
from jax import lax
from jax.experimental import pallas as pl
from jax.experimental.pallas import tpu as pltpu
from jax.experimental.pallas import tpu_sc as plsc

BATCH = 4096
N_FIELDS = 26
HIST = 20
VOCAB = 100000
DIM = 32

NC = 2
NS = 16
NW = NC * NS

NUM_BAGS = BATCH * N_FIELDS
NUM_ROWS = NUM_BAGS * HIST
BAGS_PER_W = NUM_BAGS // NW
ROWS_PER_W = BAGS_PER_W * HIST

CHUNK_BAGS = 64
CHUNK_ROWS = CHUNK_BAGS * HIST
CHUNKS_PER_W = BAGS_PER_W // CHUNK_BAGS
IDX_W = 128
N_GATHERS = CHUNK_ROWS // IDX_W


TVB = 768
TCHUNKS = VOCAB // TVB
TTAIL0 = TCHUNKS * TVB
TTAIL1 = TTAIL0 + 128
NU_A = N_FIELDS * TCHUNKS
NK_PAIRS = ((NU_A + NW - 1) // NW + 1) // 2


def _transpose_body(
    tbl_hbm, out_hbm, slab0, slab1, outb0, outb1, slab_b, outb_b, slab_c, outb_c,
    si0, si1, so0, so1,
):
    wid = lax.axis_index("s") * NC + lax.axis_index("c")
    iota = lax.iota(jnp.int32, 16)
    d_lo = iota
    d_hi = iota + 16
    rot = [(iota + r) % 16 for r in range(16)]
    oidx = [rot[r] * DIM + iota for r in range(16)]
    nu_w = (NU_A - 1 - wid) // NW + 1

    def unit_fv(k):
        u = wid + k * NW
        return u // TCHUNKS, (u % TCHUNKS) * TVB

    def fetch(k, slab, sem):
        f, v0 = unit_fv(k)
        pltpu.async_copy(tbl_hbm.at[f, :, pl.ds(v0, TVB)], slab, sem)

    def drain_in(slab, sem):
        pltpu.make_async_copy(tbl_hbm.at[0, :, pl.ds(0, TVB)], slab, sem).wait()

    def drain_out(outb, sem):
        pltpu.make_async_copy(out_hbm.at[pl.ds(0, TVB * DIM)], outb, sem).wait()

    def transpose_cols(vb, slab, outb):
        def j_body(j, carry):
            vbase = j * 16
            obase = vbase * DIM
            for r in range(16):
                vs = rot[r] + vbase
                g0 = plsc.load_gather(slab, [d_lo, vs])
                plsc.store_scatter(outb, [oidx[r] + obase], g0)
                g1 = plsc.load_gather(slab, [d_hi, vs])
                plsc.store_scatter(outb, [oidx[r] + (obase + 16)], g1)
            return carry

        lax.fori_loop(0, vb // 16, j_body, 0, unroll=4)

    def out_copy(k, outb, sem):
        f, v0 = unit_fv(k)
        e0 = pl.multiple_of((f * VOCAB + v0) * DIM, 8)
        pltpu.async_copy(outb, out_hbm.at[pl.ds(e0, TVB * DIM)], sem)

    def stage(k, slab, si, outb, so):
        @pl.when(k < nu_w)
        def _():
            drain_in(slab, si)

            @pl.when(k >= 2)
            def _():
                drain_out(outb, so)

            transpose_cols(TVB, slab, outb)
            out_copy(k, outb, so)

    @pl.when(nu_w > 0)
    def _():
        fetch(0, slab0, si0)

    def pair_body(kk, carry):
        k0 = kk * 2

        @pl.when(k0 + 1 < nu_w)
        def _():
            fetch(k0 + 1, slab1, si1)

        stage(k0, slab0, si0, outb0, so0)

        @pl.when(k0 + 2 < nu_w)
        def _():
            fetch(k0 + 2, slab0, si0)

        stage(k0 + 1, slab1, si1, outb1, so1)
        return carry

    lax.fori_loop(0, NK_PAIRS, pair_body, 0)

    @pl.when(nu_w >= 1)
    def _():
        drain_out(outb0, so0)

    @pl.when(nu_w >= 2)
    def _():
        drain_out(outb1, so1)

    @pl.when(wid < N_FIELDS)
    def _():
        for v0, vb, slab, outb in (
            (TTAIL0, 128, slab_b, outb_b),
            (TTAIL1, 32, slab_c, outb_c),
        ):
            pltpu.sync_copy(tbl_hbm.at[wid, :, pl.ds(v0, vb)], slab)
            transpose_cols(vb, slab, outb)
            e0 = pl.multiple_of((wid * VOCAB + v0) * DIM, 8)
            pltpu.sync_copy(outb, out_hbm.at[pl.ds(e0, vb * DIM)])


_transpose = pl.kernel(
    _transpose_body,
    out_type=jax.ShapeDtypeStruct((N_FIELDS * VOCAB * DIM,), jnp.float32),
    mesh=plsc.VectorSubcoreMesh(
        core_axis_name="c", subcore_axis_name="s", num_cores=NC, num_subcores=NS
    ),
    scratch_types=[
        pltpu.VMEM((DIM, TVB), jnp.float32),
        pltpu.VMEM((DIM, TVB), jnp.float32),
        pltpu.VMEM((TVB * DIM,), jnp.float32),
        pltpu.VMEM((TVB * DIM,), jnp.float32),
        pltpu.VMEM((DIM, 128), jnp.float32),
        pltpu.VMEM((128 * DIM,), jnp.float32),
        pltpu.VMEM((DIM, 32), jnp.float32),
        pltpu.VMEM((32 * DIM,), jnp.float32),
        pltpu.SemaphoreType.DMA,
        pltpu.SemaphoreType.DMA,
        pltpu.SemaphoreType.DMA,
        pltpu.SemaphoreType.DMA,
    ],
    compiler_params=pltpu.CompilerParams(needs_layout_passes=False),
)


def _lookup_body(
    table_hbm, idx_hbm, out_hbm, idx0, idx1, rows0, rows1, out_v, sem0, sem1
):
    wid = lax.axis_index("s") * NC + lax.axis_index("c")
    row_base = wid * ROWS_PER_W
    bag_base = wid * BAGS_PER_W

    def fetch(c, idx_v, sem):
        pltpu.sync_copy(idx_hbm.at[pl.ds(row_base + c * CHUNK_ROWS, CHUNK_ROWS)], idx_v)
        for j in range(N_GATHERS):
            pltpu.async_copy(
                table_hbm.at[idx_v.at[pl.ds(j * IDX_W, IDX_W)]],
                rows0.at[pl.ds(j * IDX_W, IDX_W)] if idx_v is idx0
                else rows1.at[pl.ds(j * IDX_W, IDX_W)],
                sem,
            )

    def drain(rows_v, sem):
        pltpu.make_async_copy(
            table_hbm.at[pl.ds(0, CHUNK_ROWS)], rows_v, sem
        ).wait()

    def pool_store(c, rows_v):
        def bag_body(g, carry2):
            base = g * HIST
            a0 = rows_v[base, pl.ds(0, 16)]
            a1 = rows_v[base, pl.ds(16, 16)]
            for h in range(1, HIST):
                a0 = a0 + rows_v[base + h, pl.ds(0, 16)]
                a1 = a1 + rows_v[base + h, pl.ds(16, 16)]
            out_v[g, pl.ds(0, 16)] = a0
            out_v[g, pl.ds(16, 16)] = a1
            return carry2

        lax.fori_loop(0, CHUNK_BAGS, bag_body, 0, unroll=2)
        pltpu.sync_copy(out_v, out_hbm.at[pl.ds(bag_base + c * CHUNK_BAGS, CHUNK_BAGS)])

    fetch(0, idx0, sem0)

    def pair_body(cc, carry):
        c = cc * 2
        fetch(c + 1, idx1, sem1)
        drain(rows0, sem0)
        pool_store(c, rows0)

        @pl.when(cc + 1 < CHUNKS_PER_W // 2)
        def _():
            fetch(c + 2, idx0, sem0)

        drain(rows1, sem1)
        pool_store(c + 1, rows1)
        return carry

    lax.fori_loop(0, CHUNKS_PER_W // 2, pair_body, 0)


_lookup = pl.kernel(
    _lookup_body,
    out_type=jax.ShapeDtypeStruct((NUM_BAGS, DIM), jnp.float32),
    mesh=plsc.VectorSubcoreMesh(
        core_axis_name="c", subcore_axis_name="s", num_cores=NC, num_subcores=NS
    ),
    scratch_types=[
        pltpu.VMEM((CHUNK_ROWS,), jnp.int32),
        pltpu.VMEM((CHUNK_ROWS,), jnp.int32),
        pltpu.VMEM((CHUNK_ROWS, DIM), jnp.float32),
        pltpu.VMEM((CHUNK_ROWS, DIM), jnp.float32),
        pltpu.VMEM((CHUNK_BAGS, DIM), jnp.float32),
        pltpu.SemaphoreType.DMA,
        pltpu.SemaphoreType.DMA,
    ],
    compiler_params=pltpu.CompilerParams(use_tc_tiling_on_sc=False),
)


@jax.jit
def kernel(indices, tables):
    field_off = (jnp.arange(N_FIELDS, dtype=jnp.int32) * VOCAB)[None, :, None]
    flat_idx = (indices.astype(jnp.int32) + field_off).reshape(NUM_ROWS)
    tables_t = jnp.swapaxes(tables, 1, 2)
    flat_table = _transpose(tables_t).reshape(N_FIELDS * VOCAB, DIM)
    out = _lookup(flat_table, flat_idx)
    return out.reshape(BATCH, N_FIELDS * DIM)


# speedup vs baseline: 1.1016x; 1.0004x over previous
import jax
import jax.numpy as jnp
from jax import lax
from jax.experimental import pallas as pl
from jax.experimental.pallas import tpu as pltpu
from jax.experimental.pallas import tpu_sc as plsc

BATCH = 4096
N_FIELDS = 26
HIST = 20
VOCAB = 100000
DIM = 32

NC = 2
NS = 16
NW = NC * NS

NUM_BAGS = BATCH * N_FIELDS
NUM_ROWS = NUM_BAGS * HIST
BAGS_PER_W = NUM_BAGS // NW
ROWS_PER_W = BAGS_PER_W * HIST

CHUNK_BAGS = 64
CHUNK_ROWS = CHUNK_BAGS * HIST
CHUNKS_PER_W = BAGS_PER_W // CHUNK_BAGS
IDX_W = 128
N_GATHERS = CHUNK_ROWS // IDX_W


TVB = 768
TCHUNKS = VOCAB // TVB
TTAIL0 = TCHUNKS * TVB
TTAIL1 = TTAIL0 + 128
NU_A = N_FIELDS * TCHUNKS
NK_PAIRS = ((NU_A + NW - 1) // NW + 1) // 2


def _transpose_body(
    tbl_hbm, out_hbm, slab0, slab1, outb0, outb1, slab_b, outb_b, slab_c, outb_c,
    si0, si1, so0, so1,
):
    wid = lax.axis_index("s") * NC + lax.axis_index("c")
    iota = lax.iota(jnp.int32, 16)
    d_lo = iota
    d_hi = iota + 16
    rot = [(iota + r) % 16 for r in range(16)]
    oidx = [rot[r] * DIM + iota for r in range(16)]
    nu_w = (NU_A - 1 - wid) // NW + 1

    def unit_fv(k):
        u = wid + k * NW
        return u // TCHUNKS, (u % TCHUNKS) * TVB

    def fetch(k, slab, sem):
        f, v0 = unit_fv(k)
        pltpu.async_copy(tbl_hbm.at[f, :, pl.ds(v0, TVB)], slab, sem)

    def drain_in(slab, sem):
        pltpu.make_async_copy(tbl_hbm.at[0, :, pl.ds(0, TVB)], slab, sem).wait()

    def drain_out(outb, sem):
        pltpu.make_async_copy(out_hbm.at[pl.ds(0, TVB * DIM)], outb, sem).wait()

    def transpose_cols(vb, slab, outb):
        def j_body(j, carry):
            vbase = j * 16
            obase = vbase * DIM
            for r in range(16):
                vs = rot[r] + vbase
                g0 = plsc.load_gather(slab, [d_lo, vs])
                plsc.store_scatter(outb, [oidx[r] + obase], g0)
                g1 = plsc.load_gather(slab, [d_hi, vs])
                plsc.store_scatter(outb, [oidx[r] + (obase + 16)], g1)
            return carry

        lax.fori_loop(0, vb // 16, j_body, 0, unroll=4)

    def out_copy(k, outb, sem):
        f, v0 = unit_fv(k)
        e0 = pl.multiple_of((f * VOCAB + v0) * DIM, 8)
        pltpu.async_copy(outb, out_hbm.at[pl.ds(e0, TVB * DIM)], sem)

    def stage(k, slab, si, outb, so):
        @pl.when(k < nu_w)
        def _():
            drain_in(slab, si)

            @pl.when(k >= 2)
            def _():
                drain_out(outb, so)

            transpose_cols(TVB, slab, outb)
            out_copy(k, outb, so)

    @pl.when(nu_w > 0)
    def _():
        fetch(0, slab0, si0)

    def pair_body(kk, carry):
        k0 = kk * 2

        @pl.when(k0 + 1 < nu_w)
        def _():
            fetch(k0 + 1, slab1, si1)

        stage(k0, slab0, si0, outb0, so0)

        @pl.when(k0 + 2 < nu_w)
        def _():
            fetch(k0 + 2, slab0, si0)

        stage(k0 + 1, slab1, si1, outb1, so1)
        return carry

    lax.fori_loop(0, NK_PAIRS, pair_body, 0)

    @pl.when(nu_w >= 1)
    def _():
        drain_out(outb0, so0)

    @pl.when(nu_w >= 2)
    def _():
        drain_out(outb1, so1)

    @pl.when(wid < N_FIELDS)
    def _():
        for v0, vb, slab, outb in (
            (TTAIL0, 128, slab_b, outb_b),
            (TTAIL1, 32, slab_c, outb_c),
        ):
            pltpu.sync_copy(tbl_hbm.at[wid, :, pl.ds(v0, vb)], slab)
            transpose_cols(vb, slab, outb)
            e0 = pl.multiple_of((wid * VOCAB + v0) * DIM, 8)
            pltpu.sync_copy(outb, out_hbm.at[pl.ds(e0, vb * DIM)])


_transpose = pl.kernel(
    _transpose_body,
    out_type=jax.ShapeDtypeStruct((N_FIELDS * VOCAB * DIM,), jnp.float32),
    mesh=plsc.VectorSubcoreMesh(
        core_axis_name="c", subcore_axis_name="s", num_cores=NC, num_subcores=NS
    ),
    scratch_types=[
        pltpu.VMEM((DIM, TVB), jnp.float32),
        pltpu.VMEM((DIM, TVB), jnp.float32),
        pltpu.VMEM((TVB * DIM,), jnp.float32),
        pltpu.VMEM((TVB * DIM,), jnp.float32),
        pltpu.VMEM((DIM, 128), jnp.float32),
        pltpu.VMEM((128 * DIM,), jnp.float32),
        pltpu.VMEM((DIM, 32), jnp.float32),
        pltpu.VMEM((32 * DIM,), jnp.float32),
        pltpu.SemaphoreType.DMA,
        pltpu.SemaphoreType.DMA,
        pltpu.SemaphoreType.DMA,
        pltpu.SemaphoreType.DMA,
    ],
    compiler_params=pltpu.CompilerParams(needs_layout_passes=False),
)


def _lookup_body(
    table_hbm, idx_hbm, out_hbm, idx0, idx1, rows0, rows1, out_v, sem0, sem1
):
    wid = lax.axis_index("s") * NC + lax.axis_index("c")
    row_base = wid * ROWS_PER_W
    bag_base = wid * BAGS_PER_W

    def fetch(c, idx_v, sem):
        pltpu.sync_copy(idx_hbm.at[pl.ds(row_base + c * CHUNK_ROWS, CHUNK_ROWS)], idx_v)
        for j in range(N_GATHERS):
            pltpu.async_copy(
                table_hbm.at[idx_v.at[pl.ds(j * IDX_W, IDX_W)]],
                rows0.at[pl.ds(j * IDX_W, IDX_W)] if idx_v is idx0
                else rows1.at[pl.ds(j * IDX_W, IDX_W)],
                sem,
            )

    def drain(rows_v, sem):
        pltpu.make_async_copy(
            table_hbm.at[pl.ds(0, CHUNK_ROWS)], rows_v, sem
        ).wait()

    def pool_store(c, rows_v):
        def bag_body(g, carry2):
            base = g * HIST
            a0 = rows_v[base, pl.ds(0, 16)]
            a1 = rows_v[base, pl.ds(16, 16)]
            for h in range(1, HIST):
                a0 = a0 + rows_v[base + h, pl.ds(0, 16)]
                a1 = a1 + rows_v[base + h, pl.ds(16, 16)]
            out_v[g, pl.ds(0, 16)] = a0
            out_v[g, pl.ds(16, 16)] = a1
            return carry2

        lax.fori_loop(0, CHUNK_BAGS, bag_body, 0, unroll=2)
        pltpu.sync_copy(out_v, out_hbm.at[pl.ds(bag_base + c * CHUNK_BAGS, CHUNK_BAGS)])

    fetch(0, idx0, sem0)

    def pair_body(cc, carry):
        c = cc * 2
        fetch(c + 1, idx1, sem1)
        drain(rows0, sem0)
        pool_store(c, rows0)

        @pl.when(cc + 1 < CHUNKS_PER_W // 2)
        def _():
            fetch(c + 2, idx0, sem0)

        drain(rows1, sem1)
        pool_store(c + 1, rows1)
        return carry

    lax.fori_loop(0, CHUNKS_PER_W // 2, pair_body, 0)


_lookup = pl.kernel(
    _lookup_body,
    out_type=jax.ShapeDtypeStruct((NUM_BAGS, DIM), jnp.float32),
    mesh=plsc.VectorSubcoreMesh(
        core_axis_name="c", subcore_axis_name="s", num_cores=NC, num_subcores=NS
    ),
    scratch_types=[
        pltpu.VMEM((CHUNK_ROWS,), jnp.int32),
        pltpu.VMEM((CHUNK_ROWS,), jnp.int32),
        pltpu.VMEM((CHUNK_ROWS, DIM), jnp.float32),
        pltpu.VMEM((CHUNK_ROWS, DIM), jnp.float32),
        pltpu.VMEM((CHUNK_BAGS, DIM), jnp.float32),
        pltpu.SemaphoreType.DMA,
        pltpu.SemaphoreType.DMA,
    ],
    compiler_params=pltpu.CompilerParams(use_tc_tiling_on_sc=False),
)


@jax.jit
def kernel(indices, tables):
    field_off = (jnp.arange(N_FIELDS, dtype=jnp.int32) * VOCAB)[None, :, None]
    flat_idx = (indices.astype(jnp.int32) + field_off).reshape(NUM_ROWS)
    tables_t = jnp.swapaxes(tables, 1, 2)
    flat_table = _transpose(tables_t).reshape(N_FIELDS * VOCAB, DIM)
    out = _lookup(flat_table, flat_idx)
    return out.reshape(BATCH, N_FIELDS * DIM)
